# baseline (device time: 192410 ns/iter reference)
import jax
import jax.numpy as jnp
from jax import lax
from jax.experimental import pallas as pl
from jax.experimental.pallas import tpu as pltpu

N_DEV = 4
NQ = 8


def kernel(x, w_mat, scale_x, scale_w):
    m_tot, k_loc = x.shape
    _, n_tot = w_mat.shape
    m_per = m_tot // N_DEV
    kh = k_loc // 2
    nq = n_tot // NQ

    def body(x_ref, w_ref, sx_ref, sw_ref, out_ref,
             xg, wl, wr, wd, wf8, xf8, stage, send_sems, recv_sems, copy_sems):
        q = pl.program_id(0)
        my = lax.axis_index("i")
        left = lax.rem(my + N_DEV - 1, N_DEV)
        right = lax.rem(my + 1, N_DEV)
        diag = lax.rem(my + 2, N_DEV)

        def cvt_w(t, slot):
            cp = pltpu.make_async_copy(
                w_ref.at[:, pl.ds(t * nq, nq)],
                stage.at[slot, :, pl.ds(0, nq)], copy_sems.at[slot])
            cp.start()
            cp.wait()
            wf8[:, pl.ds(t * nq, nq)] = stage[
                slot, :, pl.ds(0, nq)].astype(jnp.float8_e4m3fn)

        def cvt_x(dev, slot):
            cp = pltpu.make_async_copy(
                x_ref.at[pl.ds(dev * m_per, m_per), :],
                stage.at[slot], copy_sems.at[slot])
            cp.start()
            cp.wait()
            xf8[pl.ds(dev * m_per, m_per), :] = stage[slot].astype(
                jnp.float8_e4m3fn)

        def rc(i, src, dst, dev):
            return pltpu.make_async_remote_copy(
                src_ref=src, dst_ref=dst,
                send_sem=send_sems.at[i], recv_sem=recv_sems.at[i],
                device_id=(dev,), device_id_type=pl.DeviceIdType.MESH,
            )

        def desc_a_t(t):
            return rc(t, wf8.at[pl.ds(0, kh), pl.ds(t * nq, nq)],
                      wl.at[t, pl.ds(0, kh), :], right)

        def desc_a_b(t):
            return rc(NQ + t, wf8.at[pl.ds(kh, kh), pl.ds(t * nq, nq)],
                      wl.at[t, pl.ds(kh, kh), :], right)

        def desc_b_b(t):
            return rc(2 * NQ + t, wf8.at[pl.ds(kh, kh), pl.ds(t * nq, nq)],
                      wr.at[t, pl.ds(kh, kh), :], left)

        def desc_b_t(t):
            return rc(3 * NQ + t, wf8.at[pl.ds(0, kh), pl.ds(t * nq, nq)],
                      wr.at[t, pl.ds(0, kh), :], left)

        def desc_f(t):
            return rc(4 * NQ + t, wl.at[t, pl.ds(0, kh), :],
                      wd.at[t, pl.ds(0, kh), :], right)

        def desc_g(t):
            return rc(5 * NQ + t, wr.at[t, pl.ds(kh, kh), :],
                      wd.at[t, pl.ds(kh, kh), :], left)

        def desc_x(i, src_dev):
            return rc(6 * NQ + i, xf8.at[pl.ds(src_dev * m_per, m_per), :],
                      xg.at[my], src_dev)

        def start_p1(t):
            desc_a_t(t).start()
            desc_a_b(t).start()
            desc_b_b(t).start()
            desc_b_t(t).start()

        @pl.when(q == 0)
        def _():
            cvt_w(0, 0)
            barrier = pltpu.get_barrier_semaphore()
            for nbr in (left, right, diag):
                pl.semaphore_signal(
                    barrier, inc=1, device_id=(nbr,),
                    device_id_type=pl.DeviceIdType.MESH,
                )
            pl.semaphore_wait(barrier, 3)
            start_p1(0)
            cvt_x(right, 0)
            desc_x(0, right).start()
            cvt_x(left, 1)
            desc_x(1, left).start()
            cvt_x(diag, 0)
            desc_x(2, diag).start()
            cvt_w(1, 1)
            cvt_x(my, 0)
            xcp = pltpu.make_async_copy(
                xf8.at[pl.ds(my * m_per, m_per), :],
                xg.at[my], copy_sems.at[0])
            xcp.start()
            xcp.wait()

        @pl.when(jnp.logical_and(q >= 1, q < NQ - 1))
        def _():
            cvt_w(q + 1, q % 2)

        desc_a_t(q).wait()
        desc_f(q).start()
        desc_b_b(q).wait()
        desc_g(q).start()

        @pl.when(q < NQ - 1)
        def _():
            start_p1(q + 1)

        def dot(xs, ws):
            return lax.dot_general(
                xs, ws, (((1,), (0,)), ((), ())),
                preferred_element_type=jnp.float32,
            )

        out_ref[...] = dot(xg[my], wf8[:, pl.ds(q * nq, nq)])

        @pl.when(q == 0)
        def _():
            desc_x(0, right).wait()
            desc_x(1, left).wait()

        out_ref[...] = out_ref[...] + dot(xg[left][:, 0:kh],
                                          wl[q, pl.ds(0, kh), :])
        out_ref[...] = out_ref[...] + dot(xg[right][:, kh:k_loc],
                                          wr[q, pl.ds(kh, kh), :])
        desc_a_b(q).wait()
        out_ref[...] = out_ref[...] + dot(xg[left][:, kh:k_loc],
                                          wl[q, pl.ds(kh, kh), :])
        desc_b_t(q).wait()
        out_ref[...] = out_ref[...] + dot(xg[right][:, 0:kh],
                                          wr[q, pl.ds(0, kh), :])

        @pl.when(q == 0)
        def _():
            desc_x(2, diag).wait()

        desc_f(q).wait()
        desc_g(q).wait()
        out_ref[...] = out_ref[...] + dot(xg[diag], wd[q])

        y = out_ref[...] * (sx_ref[0] * sw_ref[0])
        out_ref[...] = y / (1.0 + jnp.exp(-jnp.clip(y, -60.0, 60.0)))

    return pl.pallas_call(
        body,
        grid=(NQ,),
        in_specs=[
            pl.BlockSpec(memory_space=pl.ANY),
            pl.BlockSpec(memory_space=pl.ANY),
            pl.BlockSpec(memory_space=pltpu.SMEM),
            pl.BlockSpec(memory_space=pltpu.SMEM),
        ],
        out_specs=pl.BlockSpec((m_per, nq), lambda q: (0, q)),
        out_shape=jax.ShapeDtypeStruct((m_per, n_tot), jnp.float32),
        scratch_shapes=[
            pltpu.VMEM((N_DEV, m_per, k_loc), jnp.float8_e4m3fn),
            pltpu.VMEM((NQ, k_loc, nq), jnp.float8_e4m3fn),
            pltpu.VMEM((NQ, k_loc, nq), jnp.float8_e4m3fn),
            pltpu.VMEM((NQ, k_loc, nq), jnp.float8_e4m3fn),
            pltpu.VMEM((k_loc, n_tot), jnp.float8_e4m3fn),
            pltpu.VMEM((m_tot, k_loc), jnp.float8_e4m3fn),
            pltpu.VMEM((2, k_loc, k_loc), jnp.float32),
            pltpu.SemaphoreType.DMA((6 * NQ + 3,)),
            pltpu.SemaphoreType.DMA((6 * NQ + 3,)),
            pltpu.SemaphoreType.DMA((2,)),
        ],
        compiler_params=pltpu.CompilerParams(
            dimension_semantics=("arbitrary",),
            collective_id=0,
            vmem_limit_bytes=60 * 1024 * 1024,
        ),
    )(x, w_mat, scale_x, scale_w)


# device time: 191320 ns/iter; 1.0057x vs baseline; 1.0057x over previous
import jax
import jax.numpy as jnp
from jax import lax
from jax.experimental import pallas as pl
from jax.experimental.pallas import tpu as pltpu

N_DEV = 4
NQ = 16


def kernel(x, w_mat, scale_x, scale_w):
    m_tot, k_loc = x.shape
    _, n_tot = w_mat.shape
    m_per = m_tot // N_DEV
    kh = k_loc // 2
    nq = n_tot // NQ

    def body(x_ref, w_ref, sx_ref, sw_ref, out_ref,
             xg, wl, wr, wd, wf8, xf8, stage, send_sems, recv_sems, copy_sems):
        q = pl.program_id(0)
        my = lax.axis_index("i")
        left = lax.rem(my + N_DEV - 1, N_DEV)
        right = lax.rem(my + 1, N_DEV)
        diag = lax.rem(my + 2, N_DEV)

        def cvt_w(t, slot):
            cp = pltpu.make_async_copy(
                w_ref.at[:, pl.ds(t * nq, nq)],
                stage.at[slot, :, pl.ds(0, nq)], copy_sems.at[slot])
            cp.start()
            cp.wait()
            wf8[:, pl.ds(t * nq, nq)] = stage[
                slot, :, pl.ds(0, nq)].astype(jnp.float8_e4m3fn)

        def cvt_x(dev, slot):
            cp = pltpu.make_async_copy(
                x_ref.at[pl.ds(dev * m_per, m_per), :],
                stage.at[slot], copy_sems.at[slot])
            cp.start()
            cp.wait()
            xf8[pl.ds(dev * m_per, m_per), :] = stage[slot].astype(
                jnp.float8_e4m3fn)

        def rc(i, src, dst, dev):
            return pltpu.make_async_remote_copy(
                src_ref=src, dst_ref=dst,
                send_sem=send_sems.at[i], recv_sem=recv_sems.at[i],
                device_id=(dev,), device_id_type=pl.DeviceIdType.MESH,
            )

        def desc_a_t(t):
            return rc(t, wf8.at[pl.ds(0, kh), pl.ds(t * nq, nq)],
                      wl.at[t, pl.ds(0, kh), :], right)

        def desc_a_b(t):
            return rc(NQ + t, wf8.at[pl.ds(kh, kh), pl.ds(t * nq, nq)],
                      wl.at[t, pl.ds(kh, kh), :], right)

        def desc_b_b(t):
            return rc(2 * NQ + t, wf8.at[pl.ds(kh, kh), pl.ds(t * nq, nq)],
                      wr.at[t, pl.ds(kh, kh), :], left)

        def desc_b_t(t):
            return rc(3 * NQ + t, wf8.at[pl.ds(0, kh), pl.ds(t * nq, nq)],
                      wr.at[t, pl.ds(0, kh), :], left)

        def desc_f(t):
            return rc(4 * NQ + t, wl.at[t, pl.ds(0, kh), :],
                      wd.at[t, pl.ds(0, kh), :], right)

        def desc_g(t):
            return rc(5 * NQ + t, wr.at[t, pl.ds(kh, kh), :],
                      wd.at[t, pl.ds(kh, kh), :], left)

        def desc_x(i, src_dev):
            return rc(6 * NQ + i, xf8.at[pl.ds(src_dev * m_per, m_per), :],
                      xg.at[my], src_dev)

        def start_p1(t):
            desc_a_t(t).start()
            desc_a_b(t).start()
            desc_b_b(t).start()
            desc_b_t(t).start()

        @pl.when(q == 0)
        def _():
            cvt_w(0, 0)
            barrier = pltpu.get_barrier_semaphore()
            for nbr in (left, right, diag):
                pl.semaphore_signal(
                    barrier, inc=1, device_id=(nbr,),
                    device_id_type=pl.DeviceIdType.MESH,
                )
            pl.semaphore_wait(barrier, 3)
            start_p1(0)
            cvt_x(right, 0)
            desc_x(0, right).start()
            cvt_x(left, 1)
            desc_x(1, left).start()
            cvt_x(diag, 0)
            desc_x(2, diag).start()
            cvt_w(1, 1)
            cvt_x(my, 0)
            xcp = pltpu.make_async_copy(
                xf8.at[pl.ds(my * m_per, m_per), :],
                xg.at[my], copy_sems.at[0])
            xcp.start()
            xcp.wait()

        @pl.when(jnp.logical_and(q >= 1, q < NQ - 1))
        def _():
            cvt_w(q + 1, q % 2)

        desc_a_t(q).wait()
        desc_f(q).start()
        desc_b_b(q).wait()
        desc_g(q).start()

        @pl.when(q < NQ - 1)
        def _():
            start_p1(q + 1)

        def dot(xs, ws):
            return lax.dot_general(
                xs, ws, (((1,), (0,)), ((), ())),
                preferred_element_type=jnp.float32,
            )

        out_ref[...] = dot(xg[my], wf8[:, pl.ds(q * nq, nq)])

        @pl.when(q == 0)
        def _():
            desc_x(0, right).wait()
            desc_x(1, left).wait()

        out_ref[...] = out_ref[...] + dot(xg[left][:, 0:kh],
                                          wl[q, pl.ds(0, kh), :])
        out_ref[...] = out_ref[...] + dot(xg[right][:, kh:k_loc],
                                          wr[q, pl.ds(kh, kh), :])
        desc_a_b(q).wait()
        out_ref[...] = out_ref[...] + dot(xg[left][:, kh:k_loc],
                                          wl[q, pl.ds(kh, kh), :])
        desc_b_t(q).wait()
        out_ref[...] = out_ref[...] + dot(xg[right][:, 0:kh],
                                          wr[q, pl.ds(0, kh), :])

        @pl.when(q == 0)
        def _():
            desc_x(2, diag).wait()

        desc_f(q).wait()
        desc_g(q).wait()
        out_ref[...] = out_ref[...] + dot(xg[diag], wd[q])

        y = out_ref[...] * (sx_ref[0] * sw_ref[0])
        out_ref[...] = y / (1.0 + jnp.exp(-jnp.clip(y, -60.0, 60.0)))

    return pl.pallas_call(
        body,
        grid=(NQ,),
        in_specs=[
            pl.BlockSpec(memory_space=pl.ANY),
            pl.BlockSpec(memory_space=pl.ANY),
            pl.BlockSpec(memory_space=pltpu.SMEM),
            pl.BlockSpec(memory_space=pltpu.SMEM),
        ],
        out_specs=pl.BlockSpec((m_per, nq), lambda q: (0, q)),
        out_shape=jax.ShapeDtypeStruct((m_per, n_tot), jnp.float32),
        scratch_shapes=[
            pltpu.VMEM((N_DEV, m_per, k_loc), jnp.float8_e4m3fn),
            pltpu.VMEM((NQ, k_loc, nq), jnp.float8_e4m3fn),
            pltpu.VMEM((NQ, k_loc, nq), jnp.float8_e4m3fn),
            pltpu.VMEM((NQ, k_loc, nq), jnp.float8_e4m3fn),
            pltpu.VMEM((k_loc, n_tot), jnp.float8_e4m3fn),
            pltpu.VMEM((m_tot, k_loc), jnp.float8_e4m3fn),
            pltpu.VMEM((2, k_loc, k_loc), jnp.float32),
            pltpu.SemaphoreType.DMA((6 * NQ + 3,)),
            pltpu.SemaphoreType.DMA((6 * NQ + 3,)),
            pltpu.SemaphoreType.DMA((2,)),
        ],
        compiler_params=pltpu.CompilerParams(
            dimension_semantics=("arbitrary",),
            collective_id=0,
            vmem_limit_bytes=60 * 1024 * 1024,
        ),
    )(x, w_mat, scale_x, scale_w)
